# baseline (speedup 1.0000x reference)
import jax
import jax.numpy as jnp
from jax import lax
from jax.experimental import pallas as pl
from jax.experimental.pallas import tpu as pltpu
from jax.experimental.pallas import tpu_sc as plsc

N = 10000
F = 128
E = 320000
NC = 2
NS = 16
NW = NC * NS
C = 128
CH = 80
NBUF = 2
EPAD = NW * CH * C
AGG_ROWS = 10240
DUMP = N
ROWS_PER_SUB = AGG_ROWS // NS


def _unpack_chunk(slab, j, out_i32, b):
    for k in range(C // 32):
        w = slab[pl.ds(j * (C // 2) + k * 16, 16)]
        out_i32[b, pl.ds(k * 32, 16)] = w & 0xFFFF
        out_i32[b, pl.ds(k * 32 + 16, 16)] = lax.shift_right_logical(w, 16)


def _sc_scatter_fn(m_hbm, src_hbm, dst_hbm, out_hbm,
                   src_v, dst_v, src_i, dst_i, rows_v, acc_sh, *gsems):
    cid = lax.axis_index("c")
    sid = lax.axis_index("s")
    wid = sid * NC + cid

    zvec = jnp.zeros((16,), jnp.float32)
    for r in range(C):
        for cc in range(F // 16):
            rows_v[0, r, pl.ds(cc * 16, 16)] = zvec
    base = sid * ROWS_PER_SUB
    for t in range(ROWS_PER_SUB // C):
        pltpu.sync_copy(rows_v.at[0], acc_sh.at[pl.ds(base + t * C, C)])
    plsc.subcore_barrier()

    pltpu.sync_copy(src_hbm.at[wid], src_v)
    pltpu.sync_copy(dst_hbm.at[wid], dst_v)

    def body(j, carry):
        _unpack_chunk(src_v, j, src_i, 0)
        pltpu.async_copy(m_hbm.at[src_i.at[0]], rows_v.at[0], gsems[0]).wait()
        _unpack_chunk(dst_v, j, dst_i, 0)
        pltpu.sync_copy(rows_v.at[0], acc_sh.at[dst_i.at[0]], add=True)
        return carry
    lax.fori_loop(0, CH, body, 0)
    plsc.subcore_barrier()

    for t in range(ROWS_PER_SUB // C):
        r0 = base + t * C
        pltpu.sync_copy(acc_sh.at[pl.ds(r0, C)], out_hbm.at[cid, pl.ds(r0, C)])


_sc_scatter = pl.kernel(
    _sc_scatter_fn,
    out_type=jax.ShapeDtypeStruct((NC, AGG_ROWS, F), jnp.float32),
    mesh=plsc.VectorSubcoreMesh(core_axis_name="c", subcore_axis_name="s"),
    scratch_types=[
        pltpu.VMEM((CH * C // 2,), jnp.int32),
        pltpu.VMEM((CH * C // 2,), jnp.int32),
        pltpu.VMEM((NBUF, C), jnp.int32),
        pltpu.VMEM((NBUF, C), jnp.int32),
        pltpu.VMEM((NBUF, C, F), jnp.float32),
        pltpu.VMEM_SHARED((AGG_ROWS, F), jnp.float32),
    ] + [pltpu.SemaphoreType.DMA] * NBUF,
)


_DN = (((1,), (0,)), ((), ()))
R = 1000
GRID = N // R


def _mm_body(h_ref, w_ref, o_ref):
    o_ref[...] = lax.dot_general(h_ref[...], w_ref[...], _DN,
                                 preferred_element_type=jnp.float32)


def _first_mm(h, w):
    return pl.pallas_call(
        _mm_body,
        grid=(GRID,),
        in_specs=[pl.BlockSpec((R, F), lambda i: (i, 0)),
                  pl.BlockSpec((F, F), lambda i: (0, 0))],
        out_specs=pl.BlockSpec((R, F), lambda i: (i, 0)),
        out_shape=jax.ShapeDtypeStruct((N, F), jnp.float32),
    )(h, w)


def _gru_core(p0_ref, p1_ref, h_ref, wih_ref, whh_ref, bih_ref, bhh_ref):
    agg = p0_ref[0] + p1_ref[0]
    h = h_ref[...]
    gi = lax.dot_general(agg, wih_ref[...], _DN,
                         preferred_element_type=jnp.float32) + bih_ref[...]
    gh = lax.dot_general(h, whh_ref[...], _DN,
                         preferred_element_type=jnp.float32) + bhh_ref[...]
    r = jax.nn.sigmoid(gi[:, :F] + gh[:, :F])
    z = jax.nn.sigmoid(gi[:, F:2 * F] + gh[:, F:2 * F])
    n = jnp.tanh(gi[:, 2 * F:] + r * gh[:, 2 * F:])
    return (1.0 - z) * n + z * h


def _gru_mid_body(p0_ref, p1_ref, h_ref, wih_ref, whh_ref, bih_ref, bhh_ref,
                  wn_ref, hnew_ref, mnext_ref):
    hn = _gru_core(p0_ref, p1_ref, h_ref, wih_ref, whh_ref, bih_ref, bhh_ref)
    hnew_ref[...] = hn
    mnext_ref[...] = lax.dot_general(hn, wn_ref[...], _DN,
                                     preferred_element_type=jnp.float32)


_PART_SPECS = [pl.BlockSpec((1, R, F), lambda i: (0, i, 0)),
               pl.BlockSpec((1, R, F), lambda i: (1, i, 0))]
_GRU_W_SPECS = [pl.BlockSpec((F, 3 * F), lambda i: (0, 0)),
                pl.BlockSpec((F, 3 * F), lambda i: (0, 0)),
                pl.BlockSpec((1, 3 * F), lambda i: (0, 0)),
                pl.BlockSpec((1, 3 * F), lambda i: (0, 0))]


def _gru_mid(parts, h, wihT, whhT, bih2, bhh2, w_next):
    return pl.pallas_call(
        _gru_mid_body,
        grid=(GRID,),
        in_specs=_PART_SPECS
        + [pl.BlockSpec((R, F), lambda i: (i, 0))]
        + _GRU_W_SPECS
        + [pl.BlockSpec((F, F), lambda i: (0, 0))],
        out_specs=[pl.BlockSpec((R, F), lambda i: (i, 0)),
                   pl.BlockSpec((R, F), lambda i: (i, 0))],
        out_shape=[jax.ShapeDtypeStruct((N, F), jnp.float32),
                   jax.ShapeDtypeStruct((N, F), jnp.float32)],
    )(parts, parts, h, wihT, whhT, bih2, bhh2, w_next)


def _gru_last_body(p0_ref, p1_ref, h_ref, wih_ref, whh_ref, bih_ref, bhh_ref,
                   lw_ref, lb_ref, out_ref):
    hn = _gru_core(p0_ref, p1_ref, h_ref, wih_ref, whh_ref, bih_ref, bhh_ref)
    hr = jnp.maximum(hn, 0.0)
    out_ref[...] = lax.dot_general(hr, lw_ref[...], _DN,
                                   preferred_element_type=jnp.float32) + lb_ref[...]


def _gru_last(parts, h, wihT, whhT, bih2, bhh2, lwT, lb2):
    return pl.pallas_call(
        _gru_last_body,
        grid=(GRID,),
        in_specs=_PART_SPECS
        + [pl.BlockSpec((R, F), lambda i: (i, 0))]
        + _GRU_W_SPECS
        + [pl.BlockSpec((F, 1), lambda i: (0, 0)),
           pl.BlockSpec((1, 1), lambda i: (0, 0))],
        out_specs=pl.BlockSpec((R, 1), lambda i: (i, 0)),
        out_shape=jax.ShapeDtypeStruct((N, 1), jnp.float32),
    )(parts, parts, h, wihT, whhT, bih2, bhh2, lwT, lb2)


def kernel(x, edge_index, weight, W_ih, W_hh, b_ih, b_hh, lin_W, lin_b):
    src = edge_index[0].astype(jnp.int32)
    dst = edge_index[1].astype(jnp.int32)
    pad = EPAD - E
    pad_src = (jnp.arange(pad, dtype=jnp.int32) * 97) % N
    pad_dst = N + (jnp.arange(pad, dtype=jnp.int32) % (AGG_ROWS - N))
    def _pack(idx):
        p = idx.reshape(NW, CH * C // 2, 2)
        return p[..., 0] | (p[..., 1] << 16)
    src_w = _pack(jnp.concatenate([src, pad_src]))
    dst_w = _pack(jnp.concatenate([dst, pad_dst]))

    wihT = W_ih.T
    whhT = W_hh.T
    bih2 = b_ih.reshape(1, 3 * F)
    bhh2 = b_hh.reshape(1, 3 * F)
    lwT = lin_W.T
    lb2 = lin_b.reshape(1, 1)

    h = x
    m = _first_mm(h, weight[0])
    for i in range(2):
        parts = _sc_scatter(m, src_w, dst_w)
        h, m = _gru_mid(parts, h, wihT, whhT, bih2, bhh2, weight[i + 1])
    parts = _sc_scatter(m, src_w, dst_w)
    return _gru_last(parts, h, wihT, whhT, bih2, bhh2, lwT, lb2)


# speedup vs baseline: 2.7919x; 2.7919x over previous
import jax
import jax.numpy as jnp
from jax import lax
from jax.experimental import pallas as pl
from jax.experimental.pallas import tpu as pltpu
from jax.experimental.pallas import tpu_sc as plsc

N = 10000
F = 128
E = 320000
NC = 2
NS = 16
NW = NC * NS
C = 104
CH = 97
NBUF = 2
EPAD = NW * CH * C
AGG_ROWS = 10112
DUMP = N
ROWS_PER_SUB = AGG_ROWS // NS


def _sc_scatter_fn(m_hbm, src_hbm, dst_hbm, out_hbm,
                   src_v, dst_v, rows_v, acc_sh, *gsems):
    cid = lax.axis_index("c")
    sid = lax.axis_index("s")
    wid = sid * NC + cid

    zvec = jnp.zeros((16,), jnp.float32)
    for r in range(C):
        for cc in range(F // 16):
            rows_v[0, r, pl.ds(cc * 16, 16)] = zvec
    base = sid * ROWS_PER_SUB
    nfull = ROWS_PER_SUB // C
    for t in range(nfull):
        pltpu.sync_copy(rows_v.at[0], acc_sh.at[pl.ds(base + t * C, C)])
    rem = ROWS_PER_SUB - nfull * C
    if rem:
        pltpu.sync_copy(rows_v.at[0, pl.ds(0, rem)],
                        acc_sh.at[pl.ds(base + nfull * C, rem)])
    plsc.subcore_barrier()

    pltpu.sync_copy(src_hbm.at[wid], src_v)
    pltpu.sync_copy(dst_hbm.at[wid], dst_v)

    def fire(j, b):
        pltpu.async_copy(
            m_hbm.at[src_v.at[pl.ds(j * C, C)]], rows_v.at[b], gsems[b])

    for b in range(NBUF):
        fire(b, b)

    def body(g, carry):
        for b in range(NBUF):
            j = g * NBUF + b
            pltpu.make_async_copy(
                m_hbm.at[src_v.at[pl.ds(j * C, C)]], rows_v.at[b],
                gsems[b]).wait()
            pltpu.sync_copy(rows_v.at[b], acc_sh.at[dst_v.at[j]], add=True)

            @pl.when(j + NBUF < CH)
            def _():
                fire(j + NBUF, b)
        return carry
    lax.fori_loop(0, CH // NBUF, body, 0)

    for j in range((CH // NBUF) * NBUF, CH):
        b = j % NBUF
        pltpu.make_async_copy(
            m_hbm.at[src_v.at[pl.ds(j * C, C)]], rows_v.at[b],
            gsems[b]).wait()
        pltpu.sync_copy(rows_v.at[b], acc_sh.at[dst_v.at[j]], add=True)
    plsc.subcore_barrier()

    for t in range(nfull):
        r0 = base + t * C
        pltpu.sync_copy(acc_sh.at[pl.ds(r0, C)], out_hbm.at[cid, pl.ds(r0, C)])
    if rem:
        r0 = base + nfull * C
        pltpu.sync_copy(acc_sh.at[pl.ds(r0, rem)],
                        out_hbm.at[cid, pl.ds(r0, rem)])


_sc_scatter = pl.kernel(
    _sc_scatter_fn,
    out_type=jax.ShapeDtypeStruct((NC, AGG_ROWS, F), jnp.float32),
    mesh=plsc.VectorSubcoreMesh(core_axis_name="c", subcore_axis_name="s"),
    scratch_types=[
        pltpu.VMEM((CH * C,), jnp.int32),
        pltpu.VMEM((CH, C), jnp.int32),
        pltpu.VMEM((NBUF, C, F), jnp.float32),
        pltpu.VMEM_SHARED((AGG_ROWS, F), jnp.float32),
    ] + [pltpu.SemaphoreType.DMA] * NBUF,
)


_DN = (((1,), (0,)), ((), ()))
R = 1000
GRID = N // R


def _mm_body(h_ref, w_ref, o_ref):
    o_ref[...] = lax.dot_general(h_ref[...], w_ref[...], _DN,
                                 preferred_element_type=jnp.float32)


def _first_mm(h, w):
    return pl.pallas_call(
        _mm_body,
        grid=(GRID,),
        in_specs=[pl.BlockSpec((R, F), lambda i: (i, 0)),
                  pl.BlockSpec((F, F), lambda i: (0, 0))],
        out_specs=pl.BlockSpec((R, F), lambda i: (i, 0)),
        out_shape=jax.ShapeDtypeStruct((N, F), jnp.float32),
    )(h, w)


def _gru_core(p0_ref, p1_ref, h_ref, wih_ref, whh_ref, bih_ref, bhh_ref):
    agg = p0_ref[0] + p1_ref[0]
    h = h_ref[...]
    gi = lax.dot_general(agg, wih_ref[...], _DN,
                         preferred_element_type=jnp.float32) + bih_ref[...]
    gh = lax.dot_general(h, whh_ref[...], _DN,
                         preferred_element_type=jnp.float32) + bhh_ref[...]
    r = jax.nn.sigmoid(gi[:, :F] + gh[:, :F])
    z = jax.nn.sigmoid(gi[:, F:2 * F] + gh[:, F:2 * F])
    n = jnp.tanh(gi[:, 2 * F:] + r * gh[:, 2 * F:])
    return (1.0 - z) * n + z * h


def _gru_mid_body(p0_ref, p1_ref, h_ref, wih_ref, whh_ref, bih_ref, bhh_ref,
                  wn_ref, hnew_ref, mnext_ref):
    hn = _gru_core(p0_ref, p1_ref, h_ref, wih_ref, whh_ref, bih_ref, bhh_ref)
    hnew_ref[...] = hn
    mnext_ref[...] = lax.dot_general(hn, wn_ref[...], _DN,
                                     preferred_element_type=jnp.float32)


_PART_SPECS = [pl.BlockSpec((1, R, F), lambda i: (0, i, 0)),
               pl.BlockSpec((1, R, F), lambda i: (1, i, 0))]
_GRU_W_SPECS = [pl.BlockSpec((F, 3 * F), lambda i: (0, 0)),
                pl.BlockSpec((F, 3 * F), lambda i: (0, 0)),
                pl.BlockSpec((1, 3 * F), lambda i: (0, 0)),
                pl.BlockSpec((1, 3 * F), lambda i: (0, 0))]


def _gru_mid(parts, h, wihT, whhT, bih2, bhh2, w_next):
    return pl.pallas_call(
        _gru_mid_body,
        grid=(GRID,),
        in_specs=_PART_SPECS
        + [pl.BlockSpec((R, F), lambda i: (i, 0))]
        + _GRU_W_SPECS
        + [pl.BlockSpec((F, F), lambda i: (0, 0))],
        out_specs=[pl.BlockSpec((R, F), lambda i: (i, 0)),
                   pl.BlockSpec((R, F), lambda i: (i, 0))],
        out_shape=[jax.ShapeDtypeStruct((N, F), jnp.float32),
                   jax.ShapeDtypeStruct((N, F), jnp.float32)],
    )(parts, parts, h, wihT, whhT, bih2, bhh2, w_next)


def _gru_last_body(p0_ref, p1_ref, h_ref, wih_ref, whh_ref, bih_ref, bhh_ref,
                   lw_ref, lb_ref, out_ref):
    hn = _gru_core(p0_ref, p1_ref, h_ref, wih_ref, whh_ref, bih_ref, bhh_ref)
    hr = jnp.maximum(hn, 0.0)
    out_ref[...] = lax.dot_general(hr, lw_ref[...], _DN,
                                   preferred_element_type=jnp.float32) + lb_ref[...]


def _gru_last(parts, h, wihT, whhT, bih2, bhh2, lwT, lb2):
    return pl.pallas_call(
        _gru_last_body,
        grid=(GRID,),
        in_specs=_PART_SPECS
        + [pl.BlockSpec((R, F), lambda i: (i, 0))]
        + _GRU_W_SPECS
        + [pl.BlockSpec((F, 1), lambda i: (0, 0)),
           pl.BlockSpec((1, 1), lambda i: (0, 0))],
        out_specs=pl.BlockSpec((R, 1), lambda i: (i, 0)),
        out_shape=jax.ShapeDtypeStruct((N, 1), jnp.float32),
    )(parts, parts, h, wihT, whhT, bih2, bhh2, lwT, lb2)


def kernel(x, edge_index, weight, W_ih, W_hh, b_ih, b_hh, lin_W, lin_b):
    src = edge_index[0].astype(jnp.int32)
    dst = edge_index[1].astype(jnp.int32)
    pad = EPAD - E
    pad_src = (jnp.arange(pad, dtype=jnp.int32) * 97) % N
    pad_dst = N + (jnp.arange(pad, dtype=jnp.int32) % (AGG_ROWS - N))
    src_w = jnp.concatenate([src, pad_src]).reshape(NW, CH * C)
    dst_w = jnp.concatenate([dst, pad_dst]).reshape(NW, CH, C)

    wihT = W_ih.T
    whhT = W_hh.T
    bih2 = b_ih.reshape(1, 3 * F)
    bhh2 = b_hh.reshape(1, 3 * F)
    lwT = lin_W.T
    lb2 = lin_b.reshape(1, 1)

    h = x
    m = _first_mm(h, weight[0])
    for i in range(2):
        parts = _sc_scatter(m, src_w, dst_w)
        h, m = _gru_mid(parts, h, wihT, whhT, bih2, bhh2, weight[i + 1])
    parts = _sc_scatter(m, src_w, dst_w)
    return _gru_last(parts, h, wihT, whhT, bih2, bhh2, lwT, lb2)


# speedup vs baseline: 2.8430x; 1.0183x over previous
import jax
import jax.numpy as jnp
from jax import lax
from jax.experimental import pallas as pl
from jax.experimental.pallas import tpu as pltpu
from jax.experimental.pallas import tpu_sc as plsc

N = 10000
F = 128
E = 320000
NC = 2
NS = 16
NW = NC * NS
C = 112
CH = 90
NBUF = 2
EPAD = NW * CH * C
AGG_ROWS = 10112
DUMP = N
ROWS_PER_SUB = AGG_ROWS // NS


def _sc_scatter_fn(m_hbm, src_hbm, dst_hbm, out_hbm,
                   src_v, dst_v, rows_v, acc_sh, *gsems):
    cid = lax.axis_index("c")
    sid = lax.axis_index("s")
    wid = sid * NC + cid

    zvec = jnp.zeros((16,), jnp.float32)
    for r in range(C):
        for cc in range(F // 16):
            rows_v[0, r, pl.ds(cc * 16, 16)] = zvec
    base = sid * ROWS_PER_SUB
    nfull = ROWS_PER_SUB // C
    for t in range(nfull):
        pltpu.sync_copy(rows_v.at[0], acc_sh.at[pl.ds(base + t * C, C)])
    rem = ROWS_PER_SUB - nfull * C
    if rem:
        pltpu.sync_copy(rows_v.at[0, pl.ds(0, rem)],
                        acc_sh.at[pl.ds(base + nfull * C, rem)])
    plsc.subcore_barrier()

    pltpu.sync_copy(src_hbm.at[wid], src_v)
    pltpu.sync_copy(dst_hbm.at[wid], dst_v)

    def fire(j, b):
        pltpu.async_copy(
            m_hbm.at[src_v.at[pl.ds(j * C, C)]], rows_v.at[b], gsems[b])

    for b in range(NBUF):
        fire(b, b)

    def body(g, carry):
        for b in range(NBUF):
            j = g * NBUF + b
            pltpu.make_async_copy(
                m_hbm.at[src_v.at[pl.ds(j * C, C)]], rows_v.at[b],
                gsems[b]).wait()
            pltpu.sync_copy(rows_v.at[b],
                            acc_sh.at[dst_v.at[pl.ds(j * C, C)]], add=True)

            @pl.when(j + NBUF < CH)
            def _():
                fire(j + NBUF, b)
        return carry
    lax.fori_loop(0, CH // NBUF, body, 0)

    for j in range((CH // NBUF) * NBUF, CH):
        b = j % NBUF
        pltpu.make_async_copy(
            m_hbm.at[src_v.at[pl.ds(j * C, C)]], rows_v.at[b],
            gsems[b]).wait()
        pltpu.sync_copy(rows_v.at[b],
                        acc_sh.at[dst_v.at[pl.ds(j * C, C)]], add=True)
    plsc.subcore_barrier()

    for t in range(nfull):
        r0 = base + t * C
        pltpu.sync_copy(acc_sh.at[pl.ds(r0, C)], out_hbm.at[cid, pl.ds(r0, C)])
    if rem:
        r0 = base + nfull * C
        pltpu.sync_copy(acc_sh.at[pl.ds(r0, rem)],
                        out_hbm.at[cid, pl.ds(r0, rem)])


_sc_scatter = pl.kernel(
    _sc_scatter_fn,
    out_type=jax.ShapeDtypeStruct((NC, AGG_ROWS, F), jnp.float32),
    mesh=plsc.VectorSubcoreMesh(core_axis_name="c", subcore_axis_name="s"),
    scratch_types=[
        pltpu.VMEM((CH * C,), jnp.int32),
        pltpu.VMEM((CH * C,), jnp.int32),
        pltpu.VMEM((NBUF, C, F), jnp.float32),
        pltpu.VMEM_SHARED((AGG_ROWS, F), jnp.float32),
    ] + [pltpu.SemaphoreType.DMA] * NBUF,
)


_DN = (((1,), (0,)), ((), ()))
R = 1000
GRID = N // R


def _mm_body(h_ref, w_ref, o_ref):
    o_ref[...] = lax.dot_general(h_ref[...], w_ref[...], _DN,
                                 preferred_element_type=jnp.float32)


def _first_mm(h, w):
    return pl.pallas_call(
        _mm_body,
        grid=(GRID,),
        in_specs=[pl.BlockSpec((R, F), lambda i: (i, 0)),
                  pl.BlockSpec((F, F), lambda i: (0, 0))],
        out_specs=pl.BlockSpec((R, F), lambda i: (i, 0)),
        out_shape=jax.ShapeDtypeStruct((N, F), jnp.float32),
    )(h, w)


def _gru_core(p0_ref, p1_ref, h_ref, wih_ref, whh_ref, bih_ref, bhh_ref):
    agg = p0_ref[0] + p1_ref[0]
    h = h_ref[...]
    gi = lax.dot_general(agg, wih_ref[...], _DN,
                         preferred_element_type=jnp.float32) + bih_ref[...]
    gh = lax.dot_general(h, whh_ref[...], _DN,
                         preferred_element_type=jnp.float32) + bhh_ref[...]
    r = jax.nn.sigmoid(gi[:, :F] + gh[:, :F])
    z = jax.nn.sigmoid(gi[:, F:2 * F] + gh[:, F:2 * F])
    n = jnp.tanh(gi[:, 2 * F:] + r * gh[:, 2 * F:])
    return (1.0 - z) * n + z * h


def _gru_mid_body(p0_ref, p1_ref, h_ref, wih_ref, whh_ref, bih_ref, bhh_ref,
                  wn_ref, hnew_ref, mnext_ref):
    hn = _gru_core(p0_ref, p1_ref, h_ref, wih_ref, whh_ref, bih_ref, bhh_ref)
    hnew_ref[...] = hn
    mnext_ref[...] = lax.dot_general(hn, wn_ref[...], _DN,
                                     preferred_element_type=jnp.float32)


_PART_SPECS = [pl.BlockSpec((1, R, F), lambda i: (0, i, 0)),
               pl.BlockSpec((1, R, F), lambda i: (1, i, 0))]
_GRU_W_SPECS = [pl.BlockSpec((F, 3 * F), lambda i: (0, 0)),
                pl.BlockSpec((F, 3 * F), lambda i: (0, 0)),
                pl.BlockSpec((1, 3 * F), lambda i: (0, 0)),
                pl.BlockSpec((1, 3 * F), lambda i: (0, 0))]


def _gru_mid(parts, h, wihT, whhT, bih2, bhh2, w_next):
    return pl.pallas_call(
        _gru_mid_body,
        grid=(GRID,),
        in_specs=_PART_SPECS
        + [pl.BlockSpec((R, F), lambda i: (i, 0))]
        + _GRU_W_SPECS
        + [pl.BlockSpec((F, F), lambda i: (0, 0))],
        out_specs=[pl.BlockSpec((R, F), lambda i: (i, 0)),
                   pl.BlockSpec((R, F), lambda i: (i, 0))],
        out_shape=[jax.ShapeDtypeStruct((N, F), jnp.float32),
                   jax.ShapeDtypeStruct((N, F), jnp.float32)],
    )(parts, parts, h, wihT, whhT, bih2, bhh2, w_next)


def _gru_last_body(p0_ref, p1_ref, h_ref, wih_ref, whh_ref, bih_ref, bhh_ref,
                   lw_ref, lb_ref, out_ref):
    hn = _gru_core(p0_ref, p1_ref, h_ref, wih_ref, whh_ref, bih_ref, bhh_ref)
    hr = jnp.maximum(hn, 0.0)
    out_ref[...] = lax.dot_general(hr, lw_ref[...], _DN,
                                   preferred_element_type=jnp.float32) + lb_ref[...]


def _gru_last(parts, h, wihT, whhT, bih2, bhh2, lwT, lb2):
    return pl.pallas_call(
        _gru_last_body,
        grid=(GRID,),
        in_specs=_PART_SPECS
        + [pl.BlockSpec((R, F), lambda i: (i, 0))]
        + _GRU_W_SPECS
        + [pl.BlockSpec((F, 1), lambda i: (0, 0)),
           pl.BlockSpec((1, 1), lambda i: (0, 0))],
        out_specs=pl.BlockSpec((R, 1), lambda i: (i, 0)),
        out_shape=jax.ShapeDtypeStruct((N, 1), jnp.float32),
    )(parts, parts, h, wihT, whhT, bih2, bhh2, lwT, lb2)


def kernel(x, edge_index, weight, W_ih, W_hh, b_ih, b_hh, lin_W, lin_b):
    src = edge_index[0].astype(jnp.int32)
    dst = edge_index[1].astype(jnp.int32)
    pad = EPAD - E
    pad_src = (jnp.arange(pad, dtype=jnp.int32) * 97) % N
    pad_dst = N + (jnp.arange(pad, dtype=jnp.int32) % (AGG_ROWS - N))
    src_w = jnp.concatenate([src, pad_src]).reshape(NW, CH * C)
    dst_w = jnp.concatenate([dst, pad_dst]).reshape(NW, CH * C)

    wihT = W_ih.T
    whhT = W_hh.T
    bih2 = b_ih.reshape(1, 3 * F)
    bhh2 = b_hh.reshape(1, 3 * F)
    lwT = lin_W.T
    lb2 = lin_b.reshape(1, 1)

    h = x
    m = _first_mm(h, weight[0])
    for i in range(2):
        parts = _sc_scatter(m, src_w, dst_w)
        h, m = _gru_mid(parts, h, wihT, whhT, bih2, bhh2, weight[i + 1])
    parts = _sc_scatter(m, src_w, dst_w)
    return _gru_last(parts, h, wihT, whhT, bih2, bhh2, lwT, lb2)


# speedup vs baseline: 3.0585x; 1.0758x over previous
import jax
import jax.numpy as jnp
from jax import lax
from jax.experimental import pallas as pl
from jax.experimental.pallas import tpu as pltpu
from jax.experimental.pallas import tpu_sc as plsc

N = 10000
F = 128
E = 320000
NC = 2
NS = 16
NW = NC * NS
C = 72
CH = 140
NBUF = 3
EPAD = NW * CH * C
AGG_ROWS = 10112
DUMP = N
ROWS_PER_SUB = AGG_ROWS // NS


def _sc_scatter_fn(m_hbm, src_hbm, dst_hbm, out_hbm,
                   src_v, dst_v, rows_v, acc_sh, *gsems):
    cid = lax.axis_index("c")
    sid = lax.axis_index("s")
    wid = sid * NC + cid

    zvec = jnp.zeros((16,), jnp.float32)
    for r in range(C):
        for cc in range(F // 16):
            rows_v[0, r, pl.ds(cc * 16, 16)] = zvec
    base = sid * ROWS_PER_SUB
    nfull = ROWS_PER_SUB // C
    for t in range(nfull):
        pltpu.sync_copy(rows_v.at[0], acc_sh.at[pl.ds(base + t * C, C)])
    rem = ROWS_PER_SUB - nfull * C
    if rem:
        pltpu.sync_copy(rows_v.at[0, pl.ds(0, rem)],
                        acc_sh.at[pl.ds(base + nfull * C, rem)])
    plsc.subcore_barrier()

    pltpu.sync_copy(src_hbm.at[wid], src_v)
    pltpu.sync_copy(dst_hbm.at[wid], dst_v)

    def fire(j, b):
        pltpu.async_copy(
            m_hbm.at[src_v.at[pl.ds(j * C, C)]], rows_v.at[b], gsems[b])

    for b in range(NBUF):
        fire(b, b)

    def body(g, carry):
        for b in range(NBUF):
            j = g * NBUF + b
            pltpu.make_async_copy(
                m_hbm.at[src_v.at[pl.ds(j * C, C)]], rows_v.at[b],
                gsems[b]).wait()
            pltpu.sync_copy(rows_v.at[b],
                            acc_sh.at[dst_v.at[pl.ds(j * C, C)]], add=True)

            @pl.when(j + NBUF < CH)
            def _():
                fire(j + NBUF, b)
        return carry
    lax.fori_loop(0, CH // NBUF, body, 0)

    for j in range((CH // NBUF) * NBUF, CH):
        b = j % NBUF
        pltpu.make_async_copy(
            m_hbm.at[src_v.at[pl.ds(j * C, C)]], rows_v.at[b],
            gsems[b]).wait()
        pltpu.sync_copy(rows_v.at[b],
                        acc_sh.at[dst_v.at[pl.ds(j * C, C)]], add=True)
    plsc.subcore_barrier()

    for t in range(nfull):
        r0 = base + t * C
        pltpu.sync_copy(acc_sh.at[pl.ds(r0, C)], out_hbm.at[cid, pl.ds(r0, C)])
    if rem:
        r0 = base + nfull * C
        pltpu.sync_copy(acc_sh.at[pl.ds(r0, rem)],
                        out_hbm.at[cid, pl.ds(r0, rem)])


_sc_scatter = pl.kernel(
    _sc_scatter_fn,
    out_type=jax.ShapeDtypeStruct((NC, AGG_ROWS, F), jnp.float32),
    mesh=plsc.VectorSubcoreMesh(core_axis_name="c", subcore_axis_name="s"),
    scratch_types=[
        pltpu.VMEM((CH * C,), jnp.int32),
        pltpu.VMEM((CH * C,), jnp.int32),
        pltpu.VMEM((NBUF, C, F), jnp.float32),
        pltpu.VMEM_SHARED((AGG_ROWS, F), jnp.float32),
    ] + [pltpu.SemaphoreType.DMA] * NBUF,
)


_DN = (((1,), (0,)), ((), ()))
R = 1000
GRID = N // R


def _mm_body(h_ref, w_ref, o_ref):
    o_ref[...] = lax.dot_general(h_ref[...], w_ref[...], _DN,
                                 preferred_element_type=jnp.float32)


def _first_mm(h, w):
    return pl.pallas_call(
        _mm_body,
        grid=(GRID,),
        in_specs=[pl.BlockSpec((R, F), lambda i: (i, 0)),
                  pl.BlockSpec((F, F), lambda i: (0, 0))],
        out_specs=pl.BlockSpec((R, F), lambda i: (i, 0)),
        out_shape=jax.ShapeDtypeStruct((N, F), jnp.float32),
    )(h, w)


def _gru_core(p0_ref, p1_ref, h_ref, wih_ref, whh_ref, bih_ref, bhh_ref):
    agg = p0_ref[0] + p1_ref[0]
    h = h_ref[...]
    gi = lax.dot_general(agg, wih_ref[...], _DN,
                         preferred_element_type=jnp.float32) + bih_ref[...]
    gh = lax.dot_general(h, whh_ref[...], _DN,
                         preferred_element_type=jnp.float32) + bhh_ref[...]
    r = jax.nn.sigmoid(gi[:, :F] + gh[:, :F])
    z = jax.nn.sigmoid(gi[:, F:2 * F] + gh[:, F:2 * F])
    n = jnp.tanh(gi[:, 2 * F:] + r * gh[:, 2 * F:])
    return (1.0 - z) * n + z * h


def _gru_mid_body(p0_ref, p1_ref, h_ref, wih_ref, whh_ref, bih_ref, bhh_ref,
                  wn_ref, hnew_ref, mnext_ref):
    hn = _gru_core(p0_ref, p1_ref, h_ref, wih_ref, whh_ref, bih_ref, bhh_ref)
    hnew_ref[...] = hn
    mnext_ref[...] = lax.dot_general(hn, wn_ref[...], _DN,
                                     preferred_element_type=jnp.float32)


_PART_SPECS = [pl.BlockSpec((1, R, F), lambda i: (0, i, 0)),
               pl.BlockSpec((1, R, F), lambda i: (1, i, 0))]
_GRU_W_SPECS = [pl.BlockSpec((F, 3 * F), lambda i: (0, 0)),
                pl.BlockSpec((F, 3 * F), lambda i: (0, 0)),
                pl.BlockSpec((1, 3 * F), lambda i: (0, 0)),
                pl.BlockSpec((1, 3 * F), lambda i: (0, 0))]


def _gru_mid(parts, h, wihT, whhT, bih2, bhh2, w_next):
    return pl.pallas_call(
        _gru_mid_body,
        grid=(GRID,),
        in_specs=_PART_SPECS
        + [pl.BlockSpec((R, F), lambda i: (i, 0))]
        + _GRU_W_SPECS
        + [pl.BlockSpec((F, F), lambda i: (0, 0))],
        out_specs=[pl.BlockSpec((R, F), lambda i: (i, 0)),
                   pl.BlockSpec((R, F), lambda i: (i, 0))],
        out_shape=[jax.ShapeDtypeStruct((N, F), jnp.float32),
                   jax.ShapeDtypeStruct((N, F), jnp.float32)],
    )(parts, parts, h, wihT, whhT, bih2, bhh2, w_next)


def _gru_last_body(p0_ref, p1_ref, h_ref, wih_ref, whh_ref, bih_ref, bhh_ref,
                   lw_ref, lb_ref, out_ref):
    hn = _gru_core(p0_ref, p1_ref, h_ref, wih_ref, whh_ref, bih_ref, bhh_ref)
    hr = jnp.maximum(hn, 0.0)
    out_ref[...] = lax.dot_general(hr, lw_ref[...], _DN,
                                   preferred_element_type=jnp.float32) + lb_ref[...]


def _gru_last(parts, h, wihT, whhT, bih2, bhh2, lwT, lb2):
    return pl.pallas_call(
        _gru_last_body,
        grid=(GRID,),
        in_specs=_PART_SPECS
        + [pl.BlockSpec((R, F), lambda i: (i, 0))]
        + _GRU_W_SPECS
        + [pl.BlockSpec((F, 1), lambda i: (0, 0)),
           pl.BlockSpec((1, 1), lambda i: (0, 0))],
        out_specs=pl.BlockSpec((R, 1), lambda i: (i, 0)),
        out_shape=jax.ShapeDtypeStruct((N, 1), jnp.float32),
    )(parts, parts, h, wihT, whhT, bih2, bhh2, lwT, lb2)


def kernel(x, edge_index, weight, W_ih, W_hh, b_ih, b_hh, lin_W, lin_b):
    src = edge_index[0].astype(jnp.int32)
    dst = edge_index[1].astype(jnp.int32)
    pad = EPAD - E
    pad_src = (jnp.arange(pad, dtype=jnp.int32) * 97) % N
    pad_dst = N + (jnp.arange(pad, dtype=jnp.int32) % (AGG_ROWS - N))
    src_w = jnp.concatenate([src, pad_src]).reshape(NW, CH * C)
    dst_w = jnp.concatenate([dst, pad_dst]).reshape(NW, CH * C)

    wihT = W_ih.T
    whhT = W_hh.T
    bih2 = b_ih.reshape(1, 3 * F)
    bhh2 = b_hh.reshape(1, 3 * F)
    lwT = lin_W.T
    lb2 = lin_b.reshape(1, 1)

    h = x
    m = _first_mm(h, weight[0])
    for i in range(2):
        parts = _sc_scatter(m, src_w, dst_w)
        h, m = _gru_mid(parts, h, wihT, whhT, bih2, bhh2, weight[i + 1])
    parts = _sc_scatter(m, src_w, dst_w)
    return _gru_last(parts, h, wihT, whhT, bih2, bhh2, lwT, lb2)
